# baseline (speedup 1.0000x reference)
import functools

import jax
import jax.numpy as jnp
from jax import lax
from jax.experimental import pallas as pl
from jax.experimental.pallas import tpu as pltpu
from jax.experimental.pallas import tpu_sc as plsc

N = 10000
E = 320000
D = 128
G = 64
T = 100

NC = 2
NS = 16
NW = NC * NS
CHUNK = 64
K = 4
IB = 40
GPB = IB // K
CH = 160
NIB = CH // IB
EPW = CH * CHUNK
EP = EPW * NW

NP = 10240
STRIPE = NP // NS
RB = 1024
NB = NP // RB


@functools.cache
def _get_deg_kernel():
    mesh = plsc.VectorSubcoreMesh(core_axis_name="c", subcore_axis_name="s")
    return functools.partial(
        pl.kernel,
        out_type=jax.ShapeDtypeStruct((NC, NP), jnp.float32),
        mesh=mesh,
        scratch_types=[
            pltpu.VMEM((CH, CHUNK), jnp.int32),
            pltpu.VMEM((CHUNK,), jnp.float32),
            pltpu.VMEM_SHARED((NP,), jnp.float32),
        ],
    )(_deg_body)


def _deg_body(dst_hbm, zrow_hbm, out_hbm, dst_v, ones_v, hist_s):
    cid = lax.axis_index("c")
    sid = lax.axis_index("s")
    wid = sid * NC + cid
    pltpu.sync_copy(dst_hbm.at[wid], dst_v)
    for k in range(CHUNK // 16):
        ones_v[pl.ds(16 * k, 16)] = jnp.ones((16,), jnp.float32)
    pltpu.sync_copy(zrow_hbm, hist_s.at[pl.ds(sid * STRIPE, STRIPE)])
    plsc.subcore_barrier()

    def body(j, carry):
        pltpu.sync_copy(ones_v, hist_s.at[dst_v.at[j]], add=True)
        return carry

    lax.fori_loop(0, CH, body, 0)
    plsc.subcore_barrier()
    pltpu.sync_copy(hist_s.at[pl.ds(sid * STRIPE, STRIPE)],
                    out_hbm.at[cid, pl.ds(sid * STRIPE, STRIPE)])


@functools.cache
def _get_edge_kernel():
    mesh = plsc.VectorSubcoreMesh(core_axis_name="c", subcore_axis_name="s")
    return functools.partial(
        pl.kernel,
        out_type=jax.ShapeDtypeStruct((NC, NP, D), jnp.float32),
        mesh=mesh,
        scratch_types=(
            [pltpu.VMEM((IB, CHUNK), jnp.int32),
             pltpu.VMEM((IB, CHUNK), jnp.int32)]
            + [pltpu.VMEM((CHUNK, D), jnp.float32)] * K
            + [pltpu.SemaphoreType.DMA] * (2 * K)
            + [pltpu.VMEM_SHARED((NP, D), jnp.float32)]
        ),
    )(_edge_body)


def _edge_body(hs_hbm, src_hbm, dst_hbm, zrows_hbm, out_hbm, *scratch):
    src_v, dst_v = scratch[0], scratch[1]
    rows = scratch[2:2 + K]
    gsem = scratch[2 + K:2 + 2 * K]
    ssem = scratch[2 + 2 * K:2 + 3 * K]
    acc_s = scratch[2 + 3 * K]

    cid = lax.axis_index("c")
    sid = lax.axis_index("s")
    wid = sid * NC + cid
    stripe = pl.ds(sid * STRIPE, STRIPE)
    pltpu.sync_copy(zrows_hbm, acc_s.at[stripe])
    plsc.subcore_barrier()

    def blk(b, carry):
        pltpu.sync_copy(src_hbm.at[wid, pl.ds(b * IB, IB)], src_v)
        pltpu.sync_copy(dst_hbm.at[wid, pl.ds(b * IB, IB)], dst_v)

        for k in range(K):
            pltpu.async_copy(hs_hbm.at[src_v.at[k]], rows[k], gsem[k])

        def grp(g, c):
            for k in range(K):
                j = g * K + k
                pltpu.make_async_copy(
                    hs_hbm.at[src_v.at[j]], rows[k], gsem[k]).wait()
                pltpu.async_copy(
                    rows[k], acc_s.at[dst_v.at[j]], ssem[k], add=True)
            for k in range(K):
                pltpu.make_async_copy(
                    rows[k], acc_s.at[dst_v.at[0]], ssem[k]).wait()

                @pl.when(g + 1 < GPB)
                def _():
                    pltpu.async_copy(
                        hs_hbm.at[src_v.at[(g + 1) * K + k]],
                        rows[k], gsem[k])
            return c

        lax.fori_loop(0, GPB, grp, 0)
        return carry

    lax.fori_loop(0, NIB, blk, 0)
    plsc.subcore_barrier()
    pltpu.sync_copy(acc_s.at[stripe], out_hbm.at[cid, stripe])


def _dinv_of(deg_ref):
    deg = deg_ref[0, :] + deg_ref[1, :] + 1.0
    return lax.rsqrt(deg)[:, None]


def _prep_body(deg_ref, x_ref, w1t_ref, hs_ref):
    dinv = _dinv_of(deg_ref)
    h = jnp.dot(x_ref[...], w1t_ref[...], preferred_element_type=jnp.float32)
    hs_ref[...] = h * dinv


def _finish_body(deg_ref, acc_ref, hs_ref, b_ref, wt_ref, out_ref):
    dinv = _dinv_of(deg_ref)
    pre = (acc_ref[0] + acc_ref[1] + hs_ref[...]) * dinv + b_ref[...]
    act = jnp.maximum(pre, 0.0)
    out_ref[...] = jnp.dot(act, wt_ref[...],
                           preferred_element_type=jnp.float32) * dinv


def _final_body(deg_ref, acc_ref, hs_ref, b_ref, seg_ref, wot_ref, bo_ref,
                out_ref, sums_ref, cnt_ref):
    i = pl.program_id(0)

    @pl.when(i == 0)
    def _():
        sums_ref[...] = jnp.zeros_like(sums_ref)
        cnt_ref[...] = jnp.zeros_like(cnt_ref)

    dinv = _dinv_of(deg_ref)
    h3 = (acc_ref[0] + acc_ref[1] + hs_ref[...]) * dinv + b_ref[...]
    seg = seg_ref[0, 0, :]
    gids = lax.broadcasted_iota(jnp.int32, (G, RB), 0)
    mask = (seg[None, :] == gids).astype(jnp.float32)
    sums_ref[...] += jnp.dot(mask, h3, preferred_element_type=jnp.float32)
    cnt_ref[...] += jnp.sum(mask, axis=1, keepdims=True)

    @pl.when(i == NB - 1)
    def _():
        pooled = sums_ref[...] / jnp.maximum(cnt_ref[...], 1.0)
        out_ref[...] = jnp.dot(pooled, wot_ref[...],
                               preferred_element_type=jnp.float32) + bo_ref[...]


def _row_spec(block):
    return pl.BlockSpec(block, lambda i: (i,) + (0,) * (len(block) - 1))


def _full_spec(shape):
    return pl.BlockSpec(shape, lambda i: (0,) * len(shape))


_deg_spec = pl.BlockSpec((2, RB), lambda i: (0, i))


def _tc_prep(deg, x_pad, w1t):
    return pl.pallas_call(
        _prep_body,
        grid=(NB,),
        in_specs=[_deg_spec, _row_spec((RB, D)), _full_spec((D, D))],
        out_specs=_row_spec((RB, D)),
        out_shape=jax.ShapeDtypeStruct((NP, D), jnp.float32),
    )(deg, x_pad, w1t)


def _tc_finish(deg, acc, hs, b, wt):
    return pl.pallas_call(
        _finish_body,
        grid=(NB,),
        in_specs=[_deg_spec, pl.BlockSpec((2, RB, D), lambda i: (0, i, 0)),
                  _row_spec((RB, D)), _full_spec((1, D)), _full_spec((D, D))],
        out_specs=_row_spec((RB, D)),
        out_shape=jax.ShapeDtypeStruct((NP, D), jnp.float32),
    )(deg, acc, hs, b, wt)


def _tc_final(deg, acc, hs, b, seg3, wot, bo):
    return pl.pallas_call(
        _final_body,
        grid=(NB,),
        in_specs=[_deg_spec, pl.BlockSpec((2, RB, D), lambda i: (0, i, 0)),
                  _row_spec((RB, D)), _full_spec((1, D)),
                  pl.BlockSpec((1, 1, RB), lambda i: (i, 0, 0)),
                  _full_spec((D, 128)), _full_spec((1, 128))],
        out_specs=_full_spec((G, 128)),
        out_shape=jax.ShapeDtypeStruct((G, 128), jnp.float32),
        scratch_shapes=[pltpu.VMEM((G, D), jnp.float32),
                        pltpu.VMEM((G, 1), jnp.float32)],
    )(deg, acc, hs, b, seg3, wot, bo)


@jax.jit
def kernel(x, edge_index, batch_seg, W1, b1, W2, b2, W3, b3, W_out, b_out):
    src = edge_index[0]
    dst = edge_index[1]
    pad = EP - E
    src_p = jnp.concatenate(
        [src, jnp.zeros((pad,), jnp.int32)]).reshape(NW, CH, CHUNK)
    dst_p = jnp.concatenate(
        [dst, jnp.full((pad,), N, jnp.int32)]).reshape(NW, CH, CHUNK)

    x_pad = jnp.zeros((NP, D), x.dtype).at[:N].set(x)
    seg3 = jnp.full((NP,), G, jnp.int32).at[:N].set(batch_seg)
    seg3 = seg3.reshape(NB, 1, RB)

    zrow = jnp.zeros((STRIPE,), jnp.float32)
    zrows = jnp.zeros((STRIPE, D), jnp.float32)

    w1t = W1.T
    w2t = W2.T
    w3t = W3.T
    wot = jnp.zeros((D, 128), jnp.float32).at[:, :T].set(W_out.T)
    bo = jnp.zeros((1, 128), jnp.float32).at[0, :T].set(b_out)

    deg = _get_deg_kernel()(dst_p, zrow)
    edge_kernel = _get_edge_kernel()

    hs1 = _tc_prep(deg, x_pad, w1t)
    acc1 = edge_kernel(hs1, src_p, dst_p, zrows)
    hs2 = _tc_finish(deg, acc1, hs1, b1.reshape(1, D), w2t)
    acc2 = edge_kernel(hs2, src_p, dst_p, zrows)
    hs3 = _tc_finish(deg, acc2, hs2, b2.reshape(1, D), w3t)
    acc3 = edge_kernel(hs3, src_p, dst_p, zrows)
    out = _tc_final(deg, acc3, hs3, b3.reshape(1, D), seg3, wot, bo)
    return out[:, :T]


# speedup vs baseline: 2.3992x; 2.3992x over previous
import functools

import jax
import jax.numpy as jnp
from jax import lax
from jax.experimental import pallas as pl
from jax.experimental.pallas import tpu as pltpu
from jax.experimental.pallas import tpu_sc as plsc

N = 10000
E = 320000
D = 128
G = 64
T = 100

NC = 2
NS = 16
NW = NC * NS
CHUNK = 64
K = 4
IB = 40
GPB = IB // K
CH = 160
NIB = CH // IB
EPW = CH * CHUNK
EP = EPW * NW

NP = 10240
STRIPE = NP // NS
RB = 1024
NB = NP // RB


@functools.cache
def _get_deg_kernel():
    mesh = plsc.VectorSubcoreMesh(core_axis_name="c", subcore_axis_name="s")
    return functools.partial(
        pl.kernel,
        out_type=jax.ShapeDtypeStruct((NC, NP), jnp.float32),
        mesh=mesh,
        scratch_types=[
            pltpu.VMEM((CH, CHUNK), jnp.int32),
            pltpu.VMEM((CHUNK,), jnp.float32),
            pltpu.VMEM_SHARED((NP,), jnp.float32),
        ],
    )(_deg_body)


def _deg_body(dst_hbm, zrow_hbm, out_hbm, dst_v, ones_v, hist_s):
    cid = lax.axis_index("c")
    sid = lax.axis_index("s")
    wid = sid * NC + cid
    pltpu.sync_copy(dst_hbm.at[wid], dst_v)
    for k in range(CHUNK // 16):
        ones_v[pl.ds(16 * k, 16)] = jnp.ones((16,), jnp.float32)
    pltpu.sync_copy(zrow_hbm, hist_s.at[pl.ds(sid * STRIPE, STRIPE)])
    plsc.subcore_barrier()

    def body(j, carry):
        pltpu.sync_copy(ones_v, hist_s.at[dst_v.at[j]], add=True)
        return carry

    lax.fori_loop(0, CH, body, 0)
    plsc.subcore_barrier()
    pltpu.sync_copy(hist_s.at[pl.ds(sid * STRIPE, STRIPE)],
                    out_hbm.at[cid, pl.ds(sid * STRIPE, STRIPE)])


@functools.cache
def _get_gather_kernel():
    mesh = plsc.VectorSubcoreMesh(core_axis_name="c", subcore_axis_name="s")
    return functools.partial(
        pl.kernel,
        out_type=jax.ShapeDtypeStruct((NW, CH, CHUNK, D), jnp.float32),
        mesh=mesh,
        scratch_types=(
            [pltpu.VMEM((IB, CHUNK), jnp.int32)]
            + [pltpu.VMEM((CHUNK, D), jnp.float32)] * K
            + [pltpu.SemaphoreType.DMA] * (2 * K)
            + [pltpu.VMEM_SHARED((NP, D), jnp.float32)]
        ),
    )(_gather_body)


def _gather_body(hs_hbm, src_hbm, out_hbm, *scratch):
    src_v = scratch[0]
    rows = scratch[1:1 + K]
    gsem = scratch[1 + K:1 + 2 * K]
    wsem = scratch[1 + 2 * K:1 + 3 * K]
    tab_s = scratch[1 + 3 * K]

    cid = lax.axis_index("c")
    sid = lax.axis_index("s")
    wid = sid * NC + cid
    stripe = pl.ds(sid * STRIPE, STRIPE)
    pltpu.sync_copy(hs_hbm.at[stripe], tab_s.at[stripe])
    plsc.subcore_barrier()

    def blk(b, carry):
        pltpu.sync_copy(src_hbm.at[wid, pl.ds(b * IB, IB)], src_v)

        for k in range(K):
            pltpu.async_copy(tab_s.at[src_v.at[k]], rows[k], gsem[k])

        def grp(g, c):
            for k in range(K):
                j = g * K + k
                pltpu.make_async_copy(
                    tab_s.at[src_v.at[j]], rows[k], gsem[k]).wait()
                pltpu.async_copy(
                    rows[k], out_hbm.at[wid, b * IB + j], wsem[k])
            for k in range(K):
                pltpu.make_async_copy(
                    rows[k], out_hbm.at[wid, 0], wsem[k]).wait()

                @pl.when(g + 1 < GPB)
                def _():
                    pltpu.async_copy(
                        tab_s.at[src_v.at[(g + 1) * K + k]],
                        rows[k], gsem[k])
            return c

        lax.fori_loop(0, GPB, grp, 0)
        return carry

    lax.fori_loop(0, NIB, blk, 0)


@functools.cache
def _get_scatter_kernel():
    mesh = plsc.VectorSubcoreMesh(core_axis_name="c", subcore_axis_name="s")
    return functools.partial(
        pl.kernel,
        out_type=jax.ShapeDtypeStruct((NC, NP, D), jnp.float32),
        mesh=mesh,
        scratch_types=(
            [pltpu.VMEM((IB, CHUNK), jnp.int32)]
            + [pltpu.VMEM((CHUNK, D), jnp.float32)] * K
            + [pltpu.SemaphoreType.DMA] * (2 * K)
            + [pltpu.VMEM_SHARED((NP, D), jnp.float32)]
        ),
    )(_scatter_body)


def _scatter_body(rows_hbm, dst_hbm, zrows_hbm, out_hbm, *scratch):
    dst_v = scratch[0]
    rows = scratch[1:1 + K]
    gsem = scratch[1 + K:1 + 2 * K]
    ssem = scratch[1 + 2 * K:1 + 3 * K]
    acc_s = scratch[1 + 3 * K]

    cid = lax.axis_index("c")
    sid = lax.axis_index("s")
    wid = sid * NC + cid
    stripe = pl.ds(sid * STRIPE, STRIPE)
    pltpu.sync_copy(zrows_hbm, acc_s.at[stripe])
    plsc.subcore_barrier()

    def blk(b, carry):
        pltpu.sync_copy(dst_hbm.at[wid, pl.ds(b * IB, IB)], dst_v)

        for k in range(K):
            pltpu.async_copy(rows_hbm.at[wid, b * IB + k], rows[k], gsem[k])

        def grp(g, c):
            for k in range(K):
                j = g * K + k
                pltpu.make_async_copy(
                    rows_hbm.at[wid, b * IB + j], rows[k], gsem[k]).wait()
                pltpu.async_copy(
                    rows[k], acc_s.at[dst_v.at[j]], ssem[k], add=True)
            for k in range(K):
                pltpu.make_async_copy(
                    rows[k], acc_s.at[dst_v.at[0]], ssem[k]).wait()

                @pl.when(g + 1 < GPB)
                def _():
                    pltpu.async_copy(
                        rows_hbm.at[wid, b * IB + (g + 1) * K + k],
                        rows[k], gsem[k])
            return c

        lax.fori_loop(0, GPB, grp, 0)
        return carry

    lax.fori_loop(0, NIB, blk, 0)
    plsc.subcore_barrier()
    pltpu.sync_copy(acc_s.at[stripe], out_hbm.at[cid, stripe])


def _dinv_of(deg_ref):
    deg = deg_ref[0, :] + deg_ref[1, :] + 1.0
    return lax.rsqrt(deg)[:, None]


def _prep_body(deg_ref, x_ref, w1t_ref, hs_ref):
    dinv = _dinv_of(deg_ref)
    h = jnp.dot(x_ref[...], w1t_ref[...], preferred_element_type=jnp.float32)
    hs_ref[...] = h * dinv


def _finish_body(deg_ref, acc_ref, hs_ref, b_ref, wt_ref, out_ref):
    dinv = _dinv_of(deg_ref)
    pre = (acc_ref[0] + acc_ref[1] + hs_ref[...]) * dinv + b_ref[...]
    act = jnp.maximum(pre, 0.0)
    out_ref[...] = jnp.dot(act, wt_ref[...],
                           preferred_element_type=jnp.float32) * dinv


def _final_body(deg_ref, acc_ref, hs_ref, b_ref, seg_ref, wot_ref, bo_ref,
                out_ref, sums_ref, cnt_ref):
    i = pl.program_id(0)

    @pl.when(i == 0)
    def _():
        sums_ref[...] = jnp.zeros_like(sums_ref)
        cnt_ref[...] = jnp.zeros_like(cnt_ref)

    dinv = _dinv_of(deg_ref)
    h3 = (acc_ref[0] + acc_ref[1] + hs_ref[...]) * dinv + b_ref[...]
    seg = seg_ref[0, 0, :]
    gids = lax.broadcasted_iota(jnp.int32, (G, RB), 0)
    mask = (seg[None, :] == gids).astype(jnp.float32)
    sums_ref[...] += jnp.dot(mask, h3, preferred_element_type=jnp.float32)
    cnt_ref[...] += jnp.sum(mask, axis=1, keepdims=True)

    @pl.when(i == NB - 1)
    def _():
        pooled = sums_ref[...] / jnp.maximum(cnt_ref[...], 1.0)
        out_ref[...] = jnp.dot(pooled, wot_ref[...],
                               preferred_element_type=jnp.float32) + bo_ref[...]


def _row_spec(block):
    return pl.BlockSpec(block, lambda i: (i,) + (0,) * (len(block) - 1))


def _full_spec(shape):
    return pl.BlockSpec(shape, lambda i: (0,) * len(shape))


_deg_spec = pl.BlockSpec((2, RB), lambda i: (0, i))


def _tc_prep(deg, x_pad, w1t):
    return pl.pallas_call(
        _prep_body,
        grid=(NB,),
        in_specs=[_deg_spec, _row_spec((RB, D)), _full_spec((D, D))],
        out_specs=_row_spec((RB, D)),
        out_shape=jax.ShapeDtypeStruct((NP, D), jnp.float32),
    )(deg, x_pad, w1t)


def _tc_finish(deg, acc, hs, b, wt):
    return pl.pallas_call(
        _finish_body,
        grid=(NB,),
        in_specs=[_deg_spec, pl.BlockSpec((2, RB, D), lambda i: (0, i, 0)),
                  _row_spec((RB, D)), _full_spec((1, D)), _full_spec((D, D))],
        out_specs=_row_spec((RB, D)),
        out_shape=jax.ShapeDtypeStruct((NP, D), jnp.float32),
    )(deg, acc, hs, b, wt)


def _tc_final(deg, acc, hs, b, seg3, wot, bo):
    return pl.pallas_call(
        _final_body,
        grid=(NB,),
        in_specs=[_deg_spec, pl.BlockSpec((2, RB, D), lambda i: (0, i, 0)),
                  _row_spec((RB, D)), _full_spec((1, D)),
                  pl.BlockSpec((1, 1, RB), lambda i: (i, 0, 0)),
                  _full_spec((D, 128)), _full_spec((1, 128))],
        out_specs=_full_spec((G, 128)),
        out_shape=jax.ShapeDtypeStruct((G, 128), jnp.float32),
        scratch_shapes=[pltpu.VMEM((G, D), jnp.float32),
                        pltpu.VMEM((G, 1), jnp.float32)],
    )(deg, acc, hs, b, seg3, wot, bo)


@jax.jit
def kernel(x, edge_index, batch_seg, W1, b1, W2, b2, W3, b3, W_out, b_out):
    src = edge_index[0]
    dst = edge_index[1]
    pad = EP - E
    src_p = jnp.concatenate(
        [src, jnp.zeros((pad,), jnp.int32)]).reshape(NW, CH, CHUNK)
    dst_p = jnp.concatenate(
        [dst, jnp.full((pad,), N, jnp.int32)]).reshape(NW, CH, CHUNK)

    x_pad = jnp.zeros((NP, D), x.dtype).at[:N].set(x)
    seg3 = jnp.full((NP,), G, jnp.int32).at[:N].set(batch_seg)
    seg3 = seg3.reshape(NB, 1, RB)

    zrow = jnp.zeros((STRIPE,), jnp.float32)
    zrows = jnp.zeros((STRIPE, D), jnp.float32)

    w1t = W1.T
    w2t = W2.T
    w3t = W3.T
    wot = jnp.zeros((D, 128), jnp.float32).at[:, :T].set(W_out.T)
    bo = jnp.zeros((1, 128), jnp.float32).at[0, :T].set(b_out)

    deg = _get_deg_kernel()(dst_p, zrow)
    gather_kernel = _get_gather_kernel()
    scatter_kernel = _get_scatter_kernel()

    def edge_pass(hs):
        rows = gather_kernel(hs, src_p)
        return scatter_kernel(rows, dst_p, zrows)

    hs1 = _tc_prep(deg, x_pad, w1t)
    acc1 = edge_pass(hs1)
    hs2 = _tc_finish(deg, acc1, hs1, b1.reshape(1, D), w2t)
    acc2 = edge_pass(hs2)
    hs3 = _tc_finish(deg, acc2, hs2, b2.reshape(1, D), w3t)
    acc3 = edge_pass(hs3)
    out = _tc_final(deg, acc3, hs3, b3.reshape(1, D), seg3, wot, bo)
    return out[:, :T]


# speedup vs baseline: 2.4154x; 1.0068x over previous
import functools

import jax
import jax.numpy as jnp
from jax import lax
from jax.experimental import pallas as pl
from jax.experimental.pallas import tpu as pltpu
from jax.experimental.pallas import tpu_sc as plsc

N = 10000
E = 320000
D = 128
G = 64
T = 100

NC = 2
NS = 16
NW = NC * NS
CHUNK = 64
K = 5
IB = 40
GPB = IB // K
CH = 160
NIB = CH // IB
EPW = CH * CHUNK
EP = EPW * NW

NP = 10240
STRIPE = NP // NS
RB = 1024
NB = NP // RB


@functools.cache
def _get_deg_kernel():
    mesh = plsc.VectorSubcoreMesh(core_axis_name="c", subcore_axis_name="s")
    return functools.partial(
        pl.kernel,
        out_type=jax.ShapeDtypeStruct((NC, NP), jnp.float32),
        mesh=mesh,
        scratch_types=[
            pltpu.VMEM((CH, CHUNK), jnp.int32),
            pltpu.VMEM((CHUNK,), jnp.float32),
            pltpu.VMEM_SHARED((NP,), jnp.float32),
        ],
    )(_deg_body)


def _deg_body(dst_hbm, zrow_hbm, out_hbm, dst_v, ones_v, hist_s):
    cid = lax.axis_index("c")
    sid = lax.axis_index("s")
    wid = sid * NC + cid
    pltpu.sync_copy(dst_hbm.at[wid], dst_v)
    for k in range(CHUNK // 16):
        ones_v[pl.ds(16 * k, 16)] = jnp.ones((16,), jnp.float32)
    pltpu.sync_copy(zrow_hbm, hist_s.at[pl.ds(sid * STRIPE, STRIPE)])
    plsc.subcore_barrier()

    def body(j, carry):
        pltpu.sync_copy(ones_v, hist_s.at[dst_v.at[j]], add=True)
        return carry

    lax.fori_loop(0, CH, body, 0)
    plsc.subcore_barrier()
    pltpu.sync_copy(hist_s.at[pl.ds(sid * STRIPE, STRIPE)],
                    out_hbm.at[cid, pl.ds(sid * STRIPE, STRIPE)])


@functools.cache
def _get_gather_kernel():
    mesh = plsc.VectorSubcoreMesh(core_axis_name="c", subcore_axis_name="s")
    return functools.partial(
        pl.kernel,
        out_type=jax.ShapeDtypeStruct((NW, CH, CHUNK, D), jnp.float32),
        mesh=mesh,
        scratch_types=(
            [pltpu.VMEM((IB, CHUNK), jnp.int32)]
            + [pltpu.VMEM((CHUNK, D), jnp.float32)] * K
            + [pltpu.SemaphoreType.DMA] * (2 * K)
            + [pltpu.VMEM_SHARED((NP, D), jnp.float32)]
        ),
    )(_gather_body)


def _gather_body(hs_hbm, src_hbm, out_hbm, *scratch):
    src_v = scratch[0]
    rows = scratch[1:1 + K]
    gsem = scratch[1 + K:1 + 2 * K]
    wsem = scratch[1 + 2 * K:1 + 3 * K]
    tab_s = scratch[1 + 3 * K]

    cid = lax.axis_index("c")
    sid = lax.axis_index("s")
    wid = sid * NC + cid
    stripe = pl.ds(sid * STRIPE, STRIPE)
    pltpu.sync_copy(hs_hbm.at[stripe], tab_s.at[stripe])
    plsc.subcore_barrier()

    def blk(b, carry):
        pltpu.sync_copy(src_hbm.at[wid, pl.ds(b * IB, IB)], src_v)

        for k in range(K):
            pltpu.async_copy(tab_s.at[src_v.at[k]], rows[k], gsem[k])

        def grp(g, c):
            for k in range(K):
                j = g * K + k
                pltpu.make_async_copy(
                    tab_s.at[src_v.at[j]], rows[k], gsem[k]).wait()
                pltpu.async_copy(
                    rows[k], out_hbm.at[wid, b * IB + j], wsem[k])
            for k in range(K):
                pltpu.make_async_copy(
                    rows[k], out_hbm.at[wid, 0], wsem[k]).wait()

                @pl.when(g + 1 < GPB)
                def _():
                    pltpu.async_copy(
                        tab_s.at[src_v.at[(g + 1) * K + k]],
                        rows[k], gsem[k])
            return c

        lax.fori_loop(0, GPB, grp, 0)
        return carry

    lax.fori_loop(0, NIB, blk, 0)


@functools.cache
def _get_scatter_kernel():
    mesh = plsc.VectorSubcoreMesh(core_axis_name="c", subcore_axis_name="s")
    return functools.partial(
        pl.kernel,
        out_type=jax.ShapeDtypeStruct((NC, NP, D), jnp.float32),
        mesh=mesh,
        scratch_types=(
            [pltpu.VMEM((IB, CHUNK), jnp.int32)]
            + [pltpu.VMEM((CHUNK, D), jnp.float32)] * K
            + [pltpu.SemaphoreType.DMA] * (2 * K)
            + [pltpu.VMEM_SHARED((NP, D), jnp.float32)]
        ),
    )(_scatter_body)


def _scatter_body(rows_hbm, dst_hbm, zrows_hbm, out_hbm, *scratch):
    dst_v = scratch[0]
    rows = scratch[1:1 + K]
    gsem = scratch[1 + K:1 + 2 * K]
    ssem = scratch[1 + 2 * K:1 + 3 * K]
    acc_s = scratch[1 + 3 * K]

    cid = lax.axis_index("c")
    sid = lax.axis_index("s")
    wid = sid * NC + cid
    stripe = pl.ds(sid * STRIPE, STRIPE)
    pltpu.sync_copy(zrows_hbm, acc_s.at[stripe])
    plsc.subcore_barrier()

    def blk(b, carry):
        pltpu.sync_copy(dst_hbm.at[wid, pl.ds(b * IB, IB)], dst_v)

        for k in range(K):
            pltpu.async_copy(rows_hbm.at[wid, b * IB + k], rows[k], gsem[k])

        def grp(g, c):
            for k in range(K):
                j = g * K + k
                pltpu.make_async_copy(
                    rows_hbm.at[wid, b * IB + j], rows[k], gsem[k]).wait()
                pltpu.async_copy(
                    rows[k], acc_s.at[dst_v.at[j]], ssem[k], add=True)
            for k in range(K):
                pltpu.make_async_copy(
                    rows[k], acc_s.at[dst_v.at[0]], ssem[k]).wait()

                @pl.when(g + 1 < GPB)
                def _():
                    pltpu.async_copy(
                        rows_hbm.at[wid, b * IB + (g + 1) * K + k],
                        rows[k], gsem[k])
            return c

        lax.fori_loop(0, GPB, grp, 0)
        return carry

    lax.fori_loop(0, NIB, blk, 0)
    plsc.subcore_barrier()
    pltpu.sync_copy(acc_s.at[stripe], out_hbm.at[cid, stripe])


def _dinv_of(deg_ref):
    deg = deg_ref[0, :] + deg_ref[1, :] + 1.0
    return lax.rsqrt(deg)[:, None]


def _prep_body(deg_ref, x_ref, w1t_ref, hs_ref):
    dinv = _dinv_of(deg_ref)
    h = jnp.dot(x_ref[...], w1t_ref[...], preferred_element_type=jnp.float32)
    hs_ref[...] = h * dinv


def _finish_body(deg_ref, acc_ref, hs_ref, b_ref, wt_ref, out_ref):
    dinv = _dinv_of(deg_ref)
    pre = (acc_ref[0] + acc_ref[1] + hs_ref[...]) * dinv + b_ref[...]
    act = jnp.maximum(pre, 0.0)
    out_ref[...] = jnp.dot(act, wt_ref[...],
                           preferred_element_type=jnp.float32) * dinv


def _final_body(deg_ref, acc_ref, hs_ref, b_ref, seg_ref, wot_ref, bo_ref,
                out_ref, sums_ref, cnt_ref):
    i = pl.program_id(0)

    @pl.when(i == 0)
    def _():
        sums_ref[...] = jnp.zeros_like(sums_ref)
        cnt_ref[...] = jnp.zeros_like(cnt_ref)

    dinv = _dinv_of(deg_ref)
    h3 = (acc_ref[0] + acc_ref[1] + hs_ref[...]) * dinv + b_ref[...]
    seg = seg_ref[0, 0, :]
    gids = lax.broadcasted_iota(jnp.int32, (G, RB), 0)
    mask = (seg[None, :] == gids).astype(jnp.float32)
    sums_ref[...] += jnp.dot(mask, h3, preferred_element_type=jnp.float32)
    cnt_ref[...] += jnp.sum(mask, axis=1, keepdims=True)

    @pl.when(i == NB - 1)
    def _():
        pooled = sums_ref[...] / jnp.maximum(cnt_ref[...], 1.0)
        out_ref[...] = jnp.dot(pooled, wot_ref[...],
                               preferred_element_type=jnp.float32) + bo_ref[...]


def _row_spec(block):
    return pl.BlockSpec(block, lambda i: (i,) + (0,) * (len(block) - 1))


def _full_spec(shape):
    return pl.BlockSpec(shape, lambda i: (0,) * len(shape))


_deg_spec = pl.BlockSpec((2, RB), lambda i: (0, i))


def _tc_prep(deg, x_pad, w1t):
    return pl.pallas_call(
        _prep_body,
        grid=(NB,),
        in_specs=[_deg_spec, _row_spec((RB, D)), _full_spec((D, D))],
        out_specs=_row_spec((RB, D)),
        out_shape=jax.ShapeDtypeStruct((NP, D), jnp.float32),
    )(deg, x_pad, w1t)


def _tc_finish(deg, acc, hs, b, wt):
    return pl.pallas_call(
        _finish_body,
        grid=(NB,),
        in_specs=[_deg_spec, pl.BlockSpec((2, RB, D), lambda i: (0, i, 0)),
                  _row_spec((RB, D)), _full_spec((1, D)), _full_spec((D, D))],
        out_specs=_row_spec((RB, D)),
        out_shape=jax.ShapeDtypeStruct((NP, D), jnp.float32),
    )(deg, acc, hs, b, wt)


def _tc_final(deg, acc, hs, b, seg3, wot, bo):
    return pl.pallas_call(
        _final_body,
        grid=(NB,),
        in_specs=[_deg_spec, pl.BlockSpec((2, RB, D), lambda i: (0, i, 0)),
                  _row_spec((RB, D)), _full_spec((1, D)),
                  pl.BlockSpec((1, 1, RB), lambda i: (i, 0, 0)),
                  _full_spec((D, 128)), _full_spec((1, 128))],
        out_specs=_full_spec((G, 128)),
        out_shape=jax.ShapeDtypeStruct((G, 128), jnp.float32),
        scratch_shapes=[pltpu.VMEM((G, D), jnp.float32),
                        pltpu.VMEM((G, 1), jnp.float32)],
    )(deg, acc, hs, b, seg3, wot, bo)


@jax.jit
def kernel(x, edge_index, batch_seg, W1, b1, W2, b2, W3, b3, W_out, b_out):
    src = edge_index[0]
    dst = edge_index[1]
    pad = EP - E
    src_p = jnp.concatenate(
        [src, jnp.zeros((pad,), jnp.int32)]).reshape(NW, CH, CHUNK)
    dst_p = jnp.concatenate(
        [dst, jnp.full((pad,), N, jnp.int32)]).reshape(NW, CH, CHUNK)

    x_pad = jnp.zeros((NP, D), x.dtype).at[:N].set(x)
    seg3 = jnp.full((NP,), G, jnp.int32).at[:N].set(batch_seg)
    seg3 = seg3.reshape(NB, 1, RB)

    zrow = jnp.zeros((STRIPE,), jnp.float32)
    zrows = jnp.zeros((STRIPE, D), jnp.float32)

    w1t = W1.T
    w2t = W2.T
    w3t = W3.T
    wot = jnp.zeros((D, 128), jnp.float32).at[:, :T].set(W_out.T)
    bo = jnp.zeros((1, 128), jnp.float32).at[0, :T].set(b_out)

    deg = _get_deg_kernel()(dst_p, zrow)
    gather_kernel = _get_gather_kernel()
    scatter_kernel = _get_scatter_kernel()

    def edge_pass(hs):
        rows = gather_kernel(hs, src_p)
        return scatter_kernel(rows, dst_p, zrows)

    hs1 = _tc_prep(deg, x_pad, w1t)
    acc1 = edge_pass(hs1)
    hs2 = _tc_finish(deg, acc1, hs1, b1.reshape(1, D), w2t)
    acc2 = edge_pass(hs2)
    hs3 = _tc_finish(deg, acc2, hs2, b2.reshape(1, D), w3t)
    acc3 = edge_pass(hs3)
    out = _tc_final(deg, acc3, hs3, b3.reshape(1, D), seg3, wot, bo)
    return out[:, :T]
